# Initial kernel scaffold; baseline (speedup 1.0000x reference)
import jax
import jax.numpy as jnp
from jax.experimental import pallas as pl


def kernel(nf, ei, ef, mp, mef, hf, W1n, W1e, a1s, a1d, b1, W2n, W2e, a2s, a2d, b2, Wm1, bm1, Wm2, bm2, Wh1, bh1, Wh2, bh2):
    raise NotImplementedError("write your pallas kernel here")


# speedup vs baseline: 1.0012x; 1.0012x over previous
import functools

import jax
import jax.numpy as jnp
from jax.experimental import pallas as pl
from jax.experimental.pallas import tpu as pltpu

HEADS = 4


def _gat_jnp(x, ei, ea, Wn, We, a_s, a_d, b):
    N = x.shape[0]
    out_dim = Wn.shape[1]
    hd = out_dim // HEADS
    Wh = (x @ Wn).reshape(N, HEADS, hd)
    s, d = ei[0], ei[1]
    e = jax.nn.leaky_relu((Wh[s] * a_s).sum(-1) + (Wh[d] * a_d).sum(-1) + ea @ We, negative_slope=0.2)
    ex = jnp.exp(e - e.max())
    den = jnp.zeros((N, HEADS), dtype=x.dtype).at[d].add(ex)
    alpha = ex / (den[d] + 1e-9)
    out = jnp.zeros((N, HEADS, hd), dtype=x.dtype).at[d].add(Wh[s] * alpha[..., None])
    return jax.nn.elu(out.reshape(N, out_dim) + b)


def _ml_body(g0_ref, g1_ref, mef_ref, Wm1_ref, bm1_ref, Wm2_ref, bm2_ref, out_ref):
    Wm1 = Wm1_ref[...]
    t = (g0_ref[...] @ Wm1[:64] + g1_ref[...] @ Wm1[64:128]
         + mef_ref[...] @ Wm1[128:134] + bm1_ref[...])
    t = jnp.maximum(t, 0.0)
    out_ref[...] = t @ Wm2_ref[...] + bm2_ref[...]


def _ml_pallas(g0, g1, mef, Wm1, bm1, Wm2, bm2):
    M = g0.shape[0]
    BLK = 2000
    grid = (M // BLK,)
    return pl.pallas_call(
        _ml_body,
        grid=grid,
        in_specs=[
            pl.BlockSpec((BLK, 64), lambda i: (i, 0)),
            pl.BlockSpec((BLK, 64), lambda i: (i, 0)),
            pl.BlockSpec((BLK, 6), lambda i: (i, 0)),
            pl.BlockSpec((134, 32), lambda i: (0, 0)),
            pl.BlockSpec((32,), lambda i: (0,)),
            pl.BlockSpec((32, 1), lambda i: (0, 0)),
            pl.BlockSpec((1,), lambda i: (0,)),
        ],
        out_specs=pl.BlockSpec((BLK, 1), lambda i: (i, 0)),
        out_shape=jax.ShapeDtypeStruct((M, 1), jnp.float32),
    )(g0, g1, mef, Wm1, bm1, Wm2, bm2)[:, 0]


def _hl_body(h_ref, hf_ref, Wh1_ref, bh1_ref, Wh2_ref, bh2_ref, out_ref):
    Wh1 = Wh1_ref[...]
    A = h_ref[...] @ Wh1[:64] + bh1_ref[...]
    B = hf_ref[...] @ Wh1[64:68]
    t = jnp.maximum(A[:, None, :] + B[None, :, :], 0.0)
    w2 = Wh2_ref[...][:, 0]
    out_ref[...] = (t * w2[None, None, :]).sum(-1) + bh2_ref[...][0]


def _hl_pallas(h, hf, Wh1, bh1, Wh2, bh2):
    N = h.shape[0]
    BLK = 1000
    grid = (N // BLK,)
    return pl.pallas_call(
        _hl_body,
        grid=grid,
        in_specs=[
            pl.BlockSpec((BLK, 64), lambda i: (i, 0)),
            pl.BlockSpec((16, 4), lambda i: (0, 0)),
            pl.BlockSpec((68, 32), lambda i: (0, 0)),
            pl.BlockSpec((32,), lambda i: (0,)),
            pl.BlockSpec((32, 1), lambda i: (0, 0)),
            pl.BlockSpec((1,), lambda i: (0,)),
        ],
        out_specs=pl.BlockSpec((BLK, 16), lambda i: (i, 0)),
        out_shape=jax.ShapeDtypeStruct((N, 16), jnp.float32),
    )(h, hf, Wh1, bh1, Wh2, bh2)


def kernel(nf, ei, ef, mp, mef, hf, W1n, W1e, a1s, a1d, b1, W2n, W2e, a2s, a2d, b2, Wm1, bm1, Wm2, bm2, Wh1, bh1, Wh2, bh2):
    h = _gat_jnp(nf, ei, ef, W1n, W1e, a1s, a1d, b1)
    h = _gat_jnp(h, ei, ef, W2n, W2e, a2s, a2d, b2)
    ml = _ml_pallas(h[mp[0]], h[mp[1]], mef, Wm1, bm1, Wm2, bm2)
    hl = _hl_pallas(h, hf, Wh1, bh1, Wh2, bh2)
    return (ml, hl)


# speedup vs baseline: 14.2677x; 14.2506x over previous
import functools

import jax
import jax.numpy as jnp
from jax import lax
from jax.experimental import pallas as pl
from jax.experimental.pallas import tpu as pltpu
from jax.experimental.pallas import tpu_sc as plsc

NC, NS, NW, L = 2, 16, 32, 16
N_ACC = 49 * 1024
NEG = -3.0e38


def _mesh():
    return plsc.VectorSubcoreMesh(core_axis_name="c", subcore_axis_name="s")


def _mm(a, b):
    return jnp.dot(a, b, precision=jax.lax.Precision.HIGHEST)


def _tc_c(ef_pad, W1e, W2e, E, E_pad):
    def body(ef_ref, w1_ref, w2_ref, c1_ref, c2_ref):
        i = pl.program_id(0)
        rid = lax.broadcasted_iota(jnp.int32, (2048, 16), 0) + i * 2048
        cid16 = lax.broadcasted_iota(jnp.int32, (2048, 16), 1)
        ok = (rid < E) & (cid16 < 4)
        ef = ef_ref[...]
        for w_ref, c_ref in ((w1_ref, c1_ref), (w2_ref, c2_ref)):
            c4 = _mm(ef, w_ref[...])
            c16 = jnp.concatenate(
                [c4, jnp.zeros((2048, 12), jnp.float32)], axis=1)
            c_ref[...] = jnp.where(ok, c16, -1e30)

    return pl.pallas_call(
        body,
        grid=(E_pad // 2048,),
        in_specs=[
            pl.BlockSpec((2048, 6), lambda i: (i, 0)),
            pl.BlockSpec((6, 4), lambda i: (0, 0)),
            pl.BlockSpec((6, 4), lambda i: (0, 0)),
        ],
        out_specs=[
            pl.BlockSpec((2048, 16), lambda i: (i, 0)),
            pl.BlockSpec((2048, 16), lambda i: (i, 0)),
        ],
        out_shape=[
            jax.ShapeDtypeStruct((E_pad, 16), jnp.float32),
            jax.ShapeDtypeStruct((E_pad, 16), jnp.float32),
        ],
    )(ef_pad, W1e, W2e)


def _mk_table(wh, ss, sd):
    z = jnp.zeros((1024, 88), jnp.float32)
    a0 = jnp.concatenate([wh[:, :32], ss, sd, z], axis=1)
    a1 = jnp.concatenate([wh[:, 32:], ss, sd, z], axis=1)
    return a0, a1


def _tc_prep(x, Wn, As, Ad):
    F = x.shape[1]

    def body(x_ref, wn_ref, as_ref, ad_ref, ta_ref, tb_ref):
        wh = _mm(x_ref[...], wn_ref[...])
        ta_ref[...], tb_ref[...] = _mk_table(wh, _mm(wh, as_ref[...]),
                                             _mm(wh, ad_ref[...]))

    return pl.pallas_call(
        body,
        grid=(N_ACC // 1024,),
        in_specs=[
            pl.BlockSpec((1024, F), lambda i: (i, 0)),
            pl.BlockSpec((F, 64), lambda i: (0, 0)),
            pl.BlockSpec((64, 4), lambda i: (0, 0)),
            pl.BlockSpec((64, 4), lambda i: (0, 0)),
        ],
        out_specs=[
            pl.BlockSpec((1024, 128), lambda i: (i, 0)),
            pl.BlockSpec((1024, 128), lambda i: (i, 0)),
        ],
        out_shape=[
            jax.ShapeDtypeStruct((N_ACC, 128), jnp.float32),
            jax.ShapeDtypeStruct((N_ACC, 128), jnp.float32),
        ],
    )(x, Wn, As, Ad)


def _elu(t):
    return jnp.where(t > 0, t, jnp.exp(jnp.minimum(t, 0.0)) - 1.0)


def _post_h(o_ref, den_ref, b_ref):
    o = o_ref[...]
    den = den_ref[...]
    rden = 1.0 / (den[0, :, :4] + den[1, :, :4] + 1e-9)
    rex = jnp.concatenate(
        [jnp.broadcast_to(rden[:, i:i + 1], (1024, 16)) for i in range(4)],
        axis=1)
    out = jnp.concatenate([o[0], o[1]], axis=-1) * rex
    return _elu(out + b_ref[...])


def _tc_post1(out2, den2, b, Wn, As, Ad):
    def body(o_ref, den_ref, b_ref, wn_ref, as_ref, ad_ref, ta_ref, tb_ref):
        h = _post_h(o_ref, den_ref, b_ref)
        wh = _mm(h, wn_ref[...])
        ta_ref[...], tb_ref[...] = _mk_table(wh, _mm(wh, as_ref[...]),
                                             _mm(wh, ad_ref[...]))

    return pl.pallas_call(
        body,
        grid=(N_ACC // 1024,),
        in_specs=[
            pl.BlockSpec((2, 1024, 32), lambda i: (0, i, 0)),
            pl.BlockSpec((2, 1024, 16), lambda i: (0, i, 0)),
            pl.BlockSpec((64,), lambda i: (0,)),
            pl.BlockSpec((64, 64), lambda i: (0, 0)),
            pl.BlockSpec((64, 4), lambda i: (0, 0)),
            pl.BlockSpec((64, 4), lambda i: (0, 0)),
        ],
        out_specs=[
            pl.BlockSpec((1024, 128), lambda i: (i, 0)),
            pl.BlockSpec((1024, 128), lambda i: (i, 0)),
        ],
        out_shape=[
            jax.ShapeDtypeStruct((N_ACC, 128), jnp.float32),
            jax.ShapeDtypeStruct((N_ACC, 128), jnp.float32),
        ],
    )(out2, den2, b, Wn, As, Ad)


def _tc_post2(out2, den2, b, Wm1, hW1):
    def body(o_ref, den_ref, b_ref, wm_ref, hw_ref, t_ref, a_ref):
        h = _post_h(o_ref, den_ref, b_ref)
        wm = wm_ref[...]
        z = jnp.zeros((1024, 64), jnp.float32)
        t_ref[...] = jnp.concatenate([_mm(h, wm[:64]), _mm(h, wm[64:128]), z], axis=1)
        a_ref[...] = _mm(h, hw_ref[...][:64])

    return pl.pallas_call(
        body,
        grid=(N_ACC // 1024,),
        in_specs=[
            pl.BlockSpec((2, 1024, 32), lambda i: (0, i, 0)),
            pl.BlockSpec((2, 1024, 16), lambda i: (0, i, 0)),
            pl.BlockSpec((64,), lambda i: (0,)),
            pl.BlockSpec((134, 32), lambda i: (0, 0)),
            pl.BlockSpec((68, 32), lambda i: (0, 0)),
        ],
        out_specs=[
            pl.BlockSpec((1024, 128), lambda i: (i, 0)),
            pl.BlockSpec((1024, 32), lambda i: (i, 0)),
        ],
        out_shape=[
            jax.ShapeDtypeStruct((N_ACC, 128), jnp.float32),
            jax.ShapeDtypeStruct((N_ACC, 32), jnp.float32),
        ],
    )(out2, den2, b, Wm1, hW1)


def _tc_ml(S, mef, Wm1, bm1, Wm2, bm2, M):
    def body(s_ref, mef_ref, wm1_ref, bm1_ref, wm2_ref, bm2_ref, o_ref):
        t = s_ref[...] + _mm(mef_ref[...], wm1_ref[...][128:134]) + bm1_ref[...]
        t = jnp.maximum(t, 0.0)
        o_ref[...] = _mm(t, wm2_ref[...]) + bm2_ref[...]

    return pl.pallas_call(
        body,
        grid=(M // 2000,),
        in_specs=[
            pl.BlockSpec((2000, 32), lambda i: (i, 0)),
            pl.BlockSpec((2000, 6), lambda i: (i, 0)),
            pl.BlockSpec((134, 32), lambda i: (0, 0)),
            pl.BlockSpec((32,), lambda i: (0,)),
            pl.BlockSpec((32, 1), lambda i: (0, 0)),
            pl.BlockSpec((1,), lambda i: (0,)),
        ],
        out_specs=pl.BlockSpec((2000, 1), lambda i: (i, 0)),
        out_shape=jax.ShapeDtypeStruct((M, 1), jnp.float32),
    )(S, mef, Wm1, bm1, Wm2, bm2)[:, 0]


def _tc_hl(A, hf, hW1, bh1, hW2, bh2, N):
    def body(a_ref, hf_ref, hw1_ref, bh1_ref, hw2_ref, bh2_ref, o_ref):
        Bm = _mm(hf_ref[...], hw1_ref[...][64:68])
        t = a_ref[...][:, None, :] + Bm[None, :, :] + bh1_ref[...]
        t = jnp.maximum(t, 0.0)
        w2 = hw2_ref[...][:, 0]
        o_ref[...] = (t * w2[None, None, :]).sum(-1) + bh2_ref[...][0]

    return pl.pallas_call(
        body,
        grid=(N // 1000,),
        in_specs=[
            pl.BlockSpec((1000, 32), lambda i: (i, 0)),
            pl.BlockSpec((16, 4), lambda i: (0, 0)),
            pl.BlockSpec((68, 32), lambda i: (0, 0)),
            pl.BlockSpec((32,), lambda i: (0,)),
            pl.BlockSpec((32, 1), lambda i: (0, 0)),
            pl.BlockSpec((1,), lambda i: (0,)),
        ],
        out_specs=pl.BlockSpec((1000, 16), lambda i: (i, 0)),
        out_shape=jax.ShapeDtypeStruct((N, 16), jnp.float32),
    )(A, hf, hW1, bh1, hW2, bh2)


def _lane_max(v):
    io = lax.iota(jnp.int32, L)
    for sh in (1, 2, 4, 8):
        v = jnp.maximum(v, jnp.take(v, io ^ sh))
    return v


def _global_max_vec(mx_hbm, mxall):
    pltpu.sync_copy(mx_hbm, mxall)

    def mr(j, mx):
        return jnp.maximum(mx, mxall[j])

    return _lane_max(lax.fori_loop(0, NW, mr,
                                   jnp.full((L,), NEG, jnp.float32)))


def _sc_e1(sd1, tab, c_flat, E_pad):
    B = 256
    CH = E_pad // (NW * B)

    @functools.partial(
        pl.kernel,
        out_type=(
            jax.ShapeDtypeStruct((E_pad * 16,), jnp.float32),
            jax.ShapeDtypeStruct((NW, L), jnp.float32),
        ),
        mesh=_mesh(),
        scratch_types=[
            pltpu.VMEM((256,), jnp.int32),
            pltpu.VMEM((256,), jnp.int32),
            pltpu.VMEM((256, 128), jnp.float32),
            pltpu.VMEM((256, 128), jnp.float32),
            pltpu.VMEM((256, 16), jnp.float32),
            pltpu.VMEM((256 * 16,), jnp.float32),
            pltpu.VMEM((L,), jnp.float32),
            pltpu.SemaphoreType.DMA,
        ],
    )
    def k(sd_hbm, t_hbm, c_hbm, e_hbm, mx_hbm,
          sbuf, dbuf, gs, gd, cbuf, ebuf, mxv, sem):
        cid = lax.axis_index("c")
        sid = lax.axis_index("s")
        wid = sid * NC + cid
        io = lax.iota(jnp.int32, L)
        shf = (io % 4) + 4

        def chunk(i, mx):
            ci = wid * CH + i
            pltpu.sync_copy(sd_hbm.at[pl.ds(ci * B, B)], sbuf)
            pltpu.sync_copy(sd_hbm.at[pl.ds(E_pad + ci * B, B)], dbuf)
            pltpu.sync_copy(c_hbm.at[pl.ds(ci * B, B)], cbuf)
            cps = [pltpu.async_copy(t_hbm.at[sbuf.at[pl.ds(j * 128, 128)]],
                                    gs.at[pl.ds(j * 128, 128)], sem)
                   for j in range(B // 128)]
            cps += [pltpu.async_copy(t_hbm.at[dbuf.at[pl.ds(j * 128, 128)]],
                                     gd.at[pl.ds(j * 128, 128)], sem)
                    for j in range(B // 128)]
            for cp in cps:
                cp.wait()

            def vec(r, mx):
                ssv = gs[r, pl.ds(32, 16)]
                sdv = gd[r, pl.ds(32, 16)]
                t = ssv + jnp.take(sdv, shf) + cbuf[r]
                e = jnp.maximum(t, 0.2 * t)
                ebuf[pl.ds(r * 16, 16)] = e
                return jnp.maximum(mx, e)

            mx = lax.fori_loop(0, B, vec, mx)
            pltpu.sync_copy(ebuf, e_hbm.at[pl.ds(ci * B * 16, B * 16)])
            return mx

        mx = lax.fori_loop(0, CH, chunk, jnp.full((L,), NEG, jnp.float32))
        mxv[...] = mx
        pltpu.sync_copy(mxv, mx_hbm.at[wid])

    return k(sd1, tab, c_flat)


def _sc_e2(sd1, e_flat, mxs, E_pad):
    B = 128
    CH = E_pad // (NW * B)
    NR = N_ACC // 8

    @functools.partial(
        pl.kernel,
        out_type=jax.ShapeDtypeStruct((NC * 2 * NR, 128), jnp.float32),
        mesh=_mesh(),
        scratch_types=[
            pltpu.VMEM((B + 16,), jnp.int32),
            pltpu.VMEM((1, B), jnp.int32),
            pltpu.VMEM((B * 16,), jnp.float32),
            pltpu.VMEM((B, 128), jnp.float32),
            pltpu.VMEM((64, 128), jnp.float32),
            pltpu.VMEM((NW, L), jnp.float32),
            pltpu.VMEM_SHARED((NR, 128), jnp.float32),
            pltpu.SemaphoreType.DMA,
        ],
    )
    def k(sd_hbm, e_hbm, mx_hbm, den_hbm,
          dbuf, didx2, ebuf, exbuf, vb, mxall, densh, sem):
        cid = lax.axis_index("c")
        sid = lax.axis_index("s")
        wid = sid * NC + cid
        zv = jnp.zeros((L,), jnp.float32)

        def zb(r, _):
            for q in range(8):
                vb[r, pl.ds(q * 16, 16)] = zv
            return 0

        lax.fori_loop(0, 64, zb, 0)

        def zb2(r, _):
            for q in range(8):
                exbuf[r, pl.ds(q * 16, 16)] = zv
            return 0

        lax.fori_loop(0, B, zb2, 0)
        RPT = NR // NS
        for kq in range(RPT // 64):
            pltpu.sync_copy(vb, densh.at[pl.ds(sid * RPT + kq * 64, 64)])
        pltpu.sync_copy(vb.at[pl.ds(0, RPT % 64)],
                        densh.at[pl.ds(sid * RPT + (RPT // 64) * 64,
                                       RPT % 64)])
        plsc.subcore_barrier()
        mv = _global_max_vec(mx_hbm, mxall)

        def chunk(i, _):
            ci = wid * CH + i
            pltpu.sync_copy(sd_hbm.at[pl.ds(E_pad + ci * B, B)],
                            dbuf.at[pl.ds(0, B)])
            pltpu.sync_copy(e_hbm.at[pl.ds(ci * B * 16, B * 16)], ebuf)
            for t in range(B // 16):
                didx2[0, pl.ds(t * 16, 16)] = dbuf[pl.ds(t * 16, 16)] >> 3

            def vec(r, _):
                dm = dbuf[pl.ds(r, 16)][0]
                off = (dm % 8) * 16
                exbuf[r, pl.ds(off, 16)] = jnp.exp(
                    ebuf[pl.ds(r * 16, 16)] - mv)
                return 0

            lax.fori_loop(0, B, vec, 0)
            pltpu.sync_copy(exbuf, densh.at[didx2.at[0]], add=True)

            def vec2(r, _):
                dm = dbuf[pl.ds(r, 16)][0]
                off = (dm % 8) * 16
                exbuf[r, pl.ds(off, 16)] = zv
                return 0

            lax.fori_loop(0, B, vec2, 0)
            return 0

        lax.fori_loop(0, CH, chunk, 0)
        plsc.subcore_barrier()
        base = cid * 2 * NR + sid * RPT
        for kq in range(RPT // 64):
            pltpu.sync_copy(densh.at[pl.ds(sid * RPT + kq * 64, 64)], vb)
            pltpu.sync_copy(vb, den_hbm.at[pl.ds(base + kq * 64, 64)])
        tl = RPT % 64
        pltpu.sync_copy(densh.at[pl.ds(sid * RPT + (RPT // 64) * 64, tl)],
                        vb.at[pl.ds(0, tl)])
        pltpu.sync_copy(vb.at[pl.ds(0, tl)],
                        den_hbm.at[pl.ds(base + (RPT // 64) * 64, tl)])

    return k(sd1, e_flat, mxs)


def _sc_e3(sd1, e_flat, mxs, tab, bc, E_pad):
    B = 64
    CH = E_pad // (NS * B)
    NR = N_ACC // 4

    @functools.partial(
        pl.kernel,
        out_type=jax.ShapeDtypeStruct((NC * NR, 128), jnp.float32),
        mesh=_mesh(),
        scratch_types=[
            pltpu.VMEM((B,), jnp.int32),
            pltpu.VMEM((B + 16,), jnp.int32),
            pltpu.VMEM((1, B), jnp.int32),
            pltpu.VMEM((B * 16,), jnp.float32),
            pltpu.VMEM((B, 128), jnp.float32),
            pltpu.VMEM((B, 128), jnp.float32),
            pltpu.VMEM((16, 128), jnp.float32),
            pltpu.VMEM((NW, L), jnp.float32),
            pltpu.VMEM((L,), jnp.int32),
            pltpu.VMEM_SHARED((NR, 128), jnp.float32),
            pltpu.SemaphoreType.DMA,
        ],
    )
    def k(sd_hbm, e_hbm, mx_hbm, t_hbm, bc_hbm, out_hbm,
          sbuf, dbuf, didx2, ebuf, gs, msg, vb, mxall, bcv, outsh, sem):
        cid = lax.axis_index("c")
        sid = lax.axis_index("s")
        zv = jnp.zeros((L,), jnp.float32)

        def zb(r, _):
            for q in range(8):
                vb[r, pl.ds(q * 16, 16)] = zv
            return 0

        lax.fori_loop(0, 16, zb, 0)

        def zb2(r, _):
            for q in range(8):
                msg[r, pl.ds(q * 16, 16)] = zv
            return 0

        lax.fori_loop(0, B, zb2, 0)
        RPT = NR // NS
        for kq in range(RPT // 16):
            pltpu.sync_copy(vb, outsh.at[pl.ds(sid * RPT + kq * 16, 16)])
        plsc.subcore_barrier()
        mv = _global_max_vec(mx_hbm, mxall)
        pltpu.sync_copy(bc_hbm.at[pl.ds(cid * 16, 16)], bcv)
        cv = bcv[...]
        ta = cv * 2
        tb = ta + 1

        def chunk(i, _):
            ci = sid * CH + i
            pltpu.sync_copy(sd_hbm.at[pl.ds(ci * B, B)], sbuf)
            pltpu.sync_copy(sd_hbm.at[pl.ds(E_pad + ci * B, B)],
                            dbuf.at[pl.ds(0, B)])
            pltpu.sync_copy(e_hbm.at[pl.ds(ci * B * 16, B * 16)], ebuf)
            for t in range(B // 16):
                sl = pl.ds(t * 16, 16)
                sbuf[sl] = sbuf[sl] + cv * N_ACC
                didx2[0, sl] = dbuf[sl] >> 2
            pltpu.async_copy(t_hbm.at[sbuf], gs, sem).wait()

            def vec(r, _):
                ex = jnp.exp(ebuf[pl.ds(r * 16, 16)] - mv)
                sa = jnp.take(ex, ta)
                sb = jnp.take(ex, tb)
                dm = dbuf[pl.ds(r, 16)][0]
                off = (dm % 4) * 32
                msg[r, pl.ds(off, 16)] = gs[r, pl.ds(0, 16)] * sa
                msg[r, pl.ds(off + 16, 16)] = gs[r, pl.ds(16, 16)] * sb
                return 0

            lax.fori_loop(0, B, vec, 0)
            pltpu.sync_copy(msg, outsh.at[didx2.at[0]], add=True)

            def vec2(r, _):
                dm = dbuf[pl.ds(r, 16)][0]
                off = (dm % 4) * 32
                msg[r, pl.ds(off, 16)] = zv
                msg[r, pl.ds(off + 16, 16)] = zv
                return 0

            lax.fori_loop(0, B, vec2, 0)
            return 0

        lax.fori_loop(0, CH, chunk, 0)
        plsc.subcore_barrier()
        base = cid * NR + sid * RPT
        for kq in range(RPT // 16):
            pltpu.sync_copy(outsh.at[pl.ds(sid * RPT + kq * 16, 16)], vb)
            pltpu.sync_copy(vb, out_hbm.at[pl.ds(base + kq * 16, 16)])

    return k(sd1, e_flat, mxs, tab, bc)


def _sc_mgather(mp0, mp1, PQ, M_pad):
    B = 256
    CH = M_pad // (NW * B)

    @functools.partial(
        pl.kernel,
        out_type=jax.ShapeDtypeStruct((M_pad * 32,), jnp.float32),
        mesh=_mesh(),
        scratch_types=[
            pltpu.VMEM((256,), jnp.int32),
            pltpu.VMEM((256,), jnp.int32),
            pltpu.VMEM((256, 128), jnp.float32),
            pltpu.VMEM((256, 128), jnp.float32),
            pltpu.VMEM((256 * 32,), jnp.float32),
            pltpu.SemaphoreType.DMA,
        ],
    )
    def k(i0_hbm, i1_hbm, t_hbm, s_hbm, i0, i1, gp, gq, sbuf, sem):
        cid = lax.axis_index("c")
        sid = lax.axis_index("s")
        wid = sid * NC + cid

        def chunk(i, _):
            ci = wid * CH + i
            pltpu.sync_copy(i0_hbm.at[pl.ds(ci * B, B)], i0)
            pltpu.sync_copy(i1_hbm.at[pl.ds(ci * B, B)], i1)
            cps = [pltpu.async_copy(t_hbm.at[i0.at[pl.ds(j * 128, 128)]],
                                    gp.at[pl.ds(j * 128, 128)], sem)
                   for j in range(B // 128)]
            cps += [pltpu.async_copy(t_hbm.at[i1.at[pl.ds(j * 128, 128)]],
                                     gq.at[pl.ds(j * 128, 128)], sem)
                    for j in range(B // 128)]
            for cp in cps:
                cp.wait()

            def vec(r, _):
                v0 = gp[r, pl.ds(0, 16)] + gq[r, pl.ds(32, 16)]
                v1 = gp[r, pl.ds(16, 16)] + gq[r, pl.ds(48, 16)]
                sbuf[pl.ds(r * 32, 16)] = v0
                sbuf[pl.ds(r * 32 + 16, 16)] = v1
                return 0

            lax.fori_loop(0, B, vec, 0)
            pltpu.sync_copy(sbuf, s_hbm.at[pl.ds(ci * B * 32, B * 32)])
            return 0

        lax.fori_loop(0, CH, chunk, 0)

    return k(mp0, mp1, PQ)


def _blockdiag(a):
    return (a[0][:, :, None] * jnp.eye(4, dtype=a.dtype)[:, None, :]).reshape(64, 4)


def kernel(nf, ei, ef, mp, mef, hf, W1n, W1e, a1s, a1d, b1, W2n, W2e, a2s,
           a2d, b2, Wm1, bm1, Wm2, bm2, Wh1, bh1, Wh2, bh2):
    N, E, M = nf.shape[0], ei.shape[1], mp.shape[1]
    CH = -(-E // (NW * 512))
    E_pad = NW * 512 * CH
    CHM = -(-M // (NW * 256))
    M_pad = NW * 256 * CHM

    pad_e = jnp.full((E_pad - E,), N, jnp.int32)
    sd1 = jnp.concatenate([ei[0], pad_e, ei[1], pad_e])
    pad_m = jnp.full((M_pad - M,), N, jnp.int32)
    mp0 = jnp.concatenate([mp[0], pad_m])
    mp1 = jnp.concatenate([mp[1], pad_m])
    ef_pad = jnp.concatenate([ef, jnp.zeros((E_pad - E, 6), jnp.float32)])
    nf_pad = jnp.zeros((N_ACC, nf.shape[1]), jnp.float32).at[:N].set(nf)
    bc = jnp.arange(NC * 16, dtype=jnp.int32) // 16

    A1s, A1d = _blockdiag(a1s), _blockdiag(a1d)
    A2s, A2d = _blockdiag(a2s), _blockdiag(a2d)

    c1, c2 = _tc_c(ef_pad, W1e, W2e, E, E_pad)

    t1a, t1b = _tc_prep(nf_pad, W1n, A1s, A1d)
    tab1 = jnp.concatenate([t1a, t1b], axis=0)
    e1, mx1 = _sc_e1(sd1, tab1, c1, E_pad)
    den1 = _sc_e2(sd1, e1, mx1, E_pad)
    out1 = _sc_e3(sd1, e1, mx1, tab1, bc, E_pad)

    t2a, t2b = _tc_post1(out1.reshape(2, N_ACC, 32),
                         den1.reshape(2, 2 * N_ACC, 16), b1, W2n, A2s, A2d)
    tab2 = jnp.concatenate([t2a, t2b], axis=0)
    e2, mx2 = _sc_e1(sd1, tab2, c2, E_pad)
    den2 = _sc_e2(sd1, e2, mx2, E_pad)
    out2 = _sc_e3(sd1, e2, mx2, tab2, bc, E_pad)

    den2v = den2.reshape(2, N_ACC // 4, 128)[:, :N_ACC // 8]
    PQ, A = _tc_post2(out2.reshape(2, N_ACC, 32),
                      den2v.reshape(2, N_ACC, 16), b2, Wm1, Wh1)
    S = _sc_mgather(mp0, mp1, PQ, M_pad)
    ml = _tc_ml(S.reshape(M_pad, 32), mef, Wm1, bm1, Wm2, bm2, M)
    hl = _tc_hl(A, hf, Wh1, bh1, Wh2, bh2, N)
    return (ml, hl)
